# baseline (speedup 1.0000x reference)
import functools

import jax
import jax.numpy as jnp
from jax import lax
from jax.experimental import pallas as pl
from jax.experimental.pallas import tpu as pltpu
from jax.experimental.pallas import tpu_sc as plsc

N_ATOMS = 1_000_000
NUM_ELEMENTS = 87
EMBED_DIM = 28
TABLE_WORDS = NUM_ELEMENTS * EMBED_DIM

CHUNK = 1536
NUM_CHUNKS = N_ATOMS // CHUNK
TAIL = N_ATOMS - NUM_CHUNKS * CHUNK
LANES = 16


def _make_sc_gather():
    info = plsc.get_sparse_core_info()
    nc, ns = info.num_cores, info.num_subcores
    nw = nc * ns
    max_t = (NUM_CHUNKS + nw - 1) // nw
    mesh = plsc.VectorSubcoreMesh(core_axis_name="c", subcore_axis_name="s")

    @functools.partial(
        pl.kernel,
        mesh=mesh,
        out_type=jax.ShapeDtypeStruct((EMBED_DIM, N_ATOMS), jnp.float32),
        scratch_types=[
            pltpu.VMEM((TABLE_WORDS,), jnp.float32),
            pltpu.VMEM((CHUNK,), jnp.int32),
            pltpu.VMEM((CHUNK,), jnp.int32),
            pltpu.VMEM((EMBED_DIM, CHUNK), jnp.float32),
            pltpu.VMEM((EMBED_DIM, CHUNK), jnp.float32),
            pltpu.VMEM((TAIL,), jnp.int32),
            pltpu.VMEM((EMBED_DIM, TAIL), jnp.float32),
            pltpu.SemaphoreType.DMA,
            pltpu.SemaphoreType.DMA,
        ],
        compiler_params=pltpu.CompilerParams(needs_layout_passes=False),
    )
    def gather_kernel(
        idx_hbm,
        table_hbm,
        out_hbm,
        table_v,
        idx0,
        idx1,
        rows0,
        rows1,
        idx_t,
        rows_t,
        sem0,
        sem1,
    ):
        wid = lax.axis_index("s") * nc + lax.axis_index("c")
        pltpu.sync_copy(table_hbm, table_v)

        def compute(idx_v, rows_v, n_atoms):
            @plsc.parallel_loop(0, n_atoms, step=LANES, unroll=4)
            def _body(off):
                at_base = idx_v[pl.ds(off, LANES)] * EMBED_DIM
                for d in range(EMBED_DIM):
                    rows_v[d, pl.ds(off, LANES)] = plsc.load_gather(
                        table_v, [at_base + d]
                    )

        def do_chunk(t, idx_v, rows_v, sem):
            chunk = wid + t * nw

            @pl.when(chunk < NUM_CHUNKS)
            def _():
                base = chunk * CHUNK
                pltpu.sync_copy(idx_hbm.at[pl.ds(base, CHUNK)], idx_v)

                @pl.when(t >= 2)
                def _():
                    pltpu.make_async_copy(
                        rows_v, out_hbm.at[:, pl.ds(0, CHUNK)], sem
                    ).wait()

                compute(idx_v, rows_v, CHUNK)
                pltpu.make_async_copy(
                    rows_v, out_hbm.at[:, pl.ds(base, CHUNK)], sem
                ).start()

        def pair_body(p, carry):
            do_chunk(2 * p, idx0, rows0, sem0)
            do_chunk(2 * p + 1, idx1, rows1, sem1)
            return carry

        lax.fori_loop(0, (max_t + 1) // 2, pair_body, 0)

        pltpu.make_async_copy(rows0, out_hbm.at[:, pl.ds(0, CHUNK)], sem0).wait()
        pltpu.make_async_copy(rows1, out_hbm.at[:, pl.ds(0, CHUNK)], sem1).wait()

        @pl.when(wid == 0)
        def _():
            base = NUM_CHUNKS * CHUNK
            pltpu.sync_copy(idx_hbm.at[pl.ds(base, TAIL)], idx_t)
            compute(idx_t, rows_t, TAIL)
            pltpu.sync_copy(rows_t, out_hbm.at[:, pl.ds(base, TAIL)])

    return gather_kernel


_gather = _make_sc_gather()


@jax.jit
def kernel(at_no, embed_ten):
    out_t = _gather(at_no.astype(jnp.int32), embed_ten.reshape(-1))
    return out_t.T


# speedup vs baseline: 1.2559x; 1.2559x over previous
import functools

import jax
import jax.numpy as jnp
from jax import lax
from jax.experimental import pallas as pl
from jax.experimental.pallas import tpu as pltpu
from jax.experimental.pallas import tpu_sc as plsc

N_ATOMS = 1_000_000
NUM_ELEMENTS = 87
EMBED_DIM = 28
TABLE_WORDS = NUM_ELEMENTS * EMBED_DIM

CHUNK = 1792
NUM_CHUNKS = N_ATOMS // CHUNK
TAIL = N_ATOMS - NUM_CHUNKS * CHUNK
LANES = 16


def _make_sc_gather():
    info = plsc.get_sparse_core_info()
    nc, ns = info.num_cores, info.num_subcores
    nw = nc * ns
    max_t = (NUM_CHUNKS + nw - 1) // nw
    mesh = plsc.VectorSubcoreMesh(core_axis_name="c", subcore_axis_name="s")

    @functools.partial(
        pl.kernel,
        mesh=mesh,
        out_type=jax.ShapeDtypeStruct((EMBED_DIM, N_ATOMS), jnp.float32),
        scratch_types=[
            pltpu.VMEM((TABLE_WORDS,), jnp.float32),
            pltpu.VMEM((CHUNK,), jnp.int32),
            pltpu.VMEM((CHUNK,), jnp.int32),
            pltpu.VMEM((EMBED_DIM, CHUNK), jnp.float32),
            pltpu.VMEM((EMBED_DIM, CHUNK), jnp.float32),
            pltpu.VMEM((TAIL,), jnp.int32),
            pltpu.VMEM((EMBED_DIM, TAIL), jnp.float32),
            pltpu.SemaphoreType.DMA,
            pltpu.SemaphoreType.DMA,
        ],
        compiler_params=pltpu.CompilerParams(needs_layout_passes=False),
    )
    def gather_kernel(
        idx_hbm,
        table_hbm,
        out_hbm,
        table_v,
        idx0,
        idx1,
        rows0,
        rows1,
        idx_t,
        rows_t,
        sem0,
        sem1,
    ):
        wid = lax.axis_index("s") * nc + lax.axis_index("c")
        pltpu.sync_copy(table_hbm, table_v)

        def compute(idx_v, rows_v, n_atoms):
            @plsc.parallel_loop(0, n_atoms, step=LANES, unroll=2)
            def _body(off):
                at_base = idx_v[pl.ds(off, LANES)] * EMBED_DIM
                for d in range(EMBED_DIM):
                    rows_v[d, pl.ds(off, LANES)] = plsc.load_gather(
                        table_v, [at_base + d]
                    )

        def do_chunk(t, idx_v, rows_v, sem):
            chunk = wid + t * nw

            @pl.when(chunk < NUM_CHUNKS)
            def _():
                base = chunk * CHUNK
                pltpu.sync_copy(idx_hbm.at[pl.ds(base, CHUNK)], idx_v)

                @pl.when(t >= 2)
                def _():
                    pltpu.make_async_copy(
                        rows_v, out_hbm.at[:, pl.ds(0, CHUNK)], sem
                    ).wait()

                compute(idx_v, rows_v, CHUNK)
                pltpu.make_async_copy(
                    rows_v, out_hbm.at[:, pl.ds(base, CHUNK)], sem
                ).start()

        def pair_body(p, carry):
            do_chunk(2 * p, idx0, rows0, sem0)
            do_chunk(2 * p + 1, idx1, rows1, sem1)
            return carry

        lax.fori_loop(0, (max_t + 1) // 2, pair_body, 0)

        pltpu.make_async_copy(rows0, out_hbm.at[:, pl.ds(0, CHUNK)], sem0).wait()
        pltpu.make_async_copy(rows1, out_hbm.at[:, pl.ds(0, CHUNK)], sem1).wait()

        @pl.when(wid == 0)
        def _():
            base = NUM_CHUNKS * CHUNK
            pltpu.sync_copy(idx_hbm.at[pl.ds(base, TAIL)], idx_t)
            compute(idx_t, rows_t, TAIL)
            pltpu.sync_copy(rows_t, out_hbm.at[:, pl.ds(base, TAIL)])

    return gather_kernel


_gather = _make_sc_gather()


@jax.jit
def kernel(at_no, embed_ten):
    out_t = _gather(at_no.astype(jnp.int32), embed_ten.reshape(-1))
    return out_t.T


# speedup vs baseline: 1.4508x; 1.1552x over previous
import functools

import jax
import jax.numpy as jnp
from jax import lax
from jax.experimental import pallas as pl
from jax.experimental.pallas import tpu as pltpu
from jax.experimental.pallas import tpu_sc as plsc

N_ATOMS = 1_000_000
NUM_ELEMENTS = 87
EMBED_DIM = 28
TABLE_WORDS = NUM_ELEMENTS * EMBED_DIM

CHUNK = 1792
NUM_CHUNKS = N_ATOMS // CHUNK
TAIL = N_ATOMS - NUM_CHUNKS * CHUNK
LANES = 16


def _make_sc_gather():
    info = plsc.get_sparse_core_info()
    nc, ns = info.num_cores, info.num_subcores
    nw = nc * ns
    max_t = (NUM_CHUNKS + nw - 1) // nw
    mesh = plsc.VectorSubcoreMesh(core_axis_name="c", subcore_axis_name="s")

    @functools.partial(
        pl.kernel,
        mesh=mesh,
        out_type=jax.ShapeDtypeStruct((EMBED_DIM, N_ATOMS), jnp.float32),
        scratch_types=[
            pltpu.VMEM((TABLE_WORDS,), jnp.float32),
            [pltpu.VMEM((CHUNK,), jnp.int32) for _ in range(4)],
            [pltpu.VMEM((EMBED_DIM, CHUNK), jnp.float32) for _ in range(2)],
            pltpu.VMEM((TAIL,), jnp.int32),
            pltpu.VMEM((EMBED_DIM, TAIL), jnp.float32),
            [pltpu.SemaphoreType.DMA for _ in range(4)],
            [pltpu.SemaphoreType.DMA for _ in range(2)],
        ],
        compiler_params=pltpu.CompilerParams(needs_layout_passes=False),
    )
    def gather_kernel(
        idx_hbm, table_hbm, out_hbm, table_v, idx, rows, idx_t, rows_t, isem, osem
    ):
        wid = lax.axis_index("s") * nc + lax.axis_index("c")

        def idx_start(slot, chunk):
            pltpu.make_async_copy(
                idx_hbm.at[pl.ds(chunk * CHUNK, CHUNK)], idx[slot], isem[slot]
            ).start()

        def idx_wait(slot):
            pltpu.make_async_copy(
                idx_hbm.at[pl.ds(0, CHUNK)], idx[slot], isem[slot]
            ).wait()

        for t0 in (0, 1):
            c0 = wid + t0 * nw

            @pl.when(c0 < NUM_CHUNKS)
            def _(t0=t0, c0=c0):
                idx_start(t0, c0)

        pltpu.sync_copy(table_hbm, table_v)

        def compute(idx_v, rows_v, n_atoms):
            @plsc.parallel_loop(0, n_atoms, step=LANES, unroll=2)
            def _body(off):
                at_base = idx_v[pl.ds(off, LANES)] * EMBED_DIM
                for d in range(EMBED_DIM):
                    rows_v[d, pl.ds(off, LANES)] = plsc.load_gather(
                        table_v, [at_base + d]
                    )

        def do_chunk(q, j):
            t = 4 * q + j
            chunk = wid + t * nw

            @pl.when(chunk < NUM_CHUNKS)
            def _():
                chunk2 = chunk + 2 * nw

                @pl.when(chunk2 < NUM_CHUNKS)
                def _():
                    idx_start((j + 2) % 4, chunk2)

                idx_wait(j)

                @pl.when(t >= 2)
                def _():
                    pltpu.make_async_copy(
                        rows[j % 2], out_hbm.at[:, pl.ds(0, CHUNK)], osem[j % 2]
                    ).wait()

                compute(idx[j], rows[j % 2], CHUNK)
                pltpu.make_async_copy(
                    rows[j % 2], out_hbm.at[:, pl.ds(chunk * CHUNK, CHUNK)], osem[j % 2]
                ).start()

        def quad_body(q, carry):
            for j in range(4):
                do_chunk(q, j)
            return carry

        lax.fori_loop(0, (max_t + 3) // 4, quad_body, 0)

        pltpu.make_async_copy(rows[0], out_hbm.at[:, pl.ds(0, CHUNK)], osem[0]).wait()
        pltpu.make_async_copy(rows[1], out_hbm.at[:, pl.ds(0, CHUNK)], osem[1]).wait()

        @pl.when(wid == 0)
        def _():
            base = NUM_CHUNKS * CHUNK
            pltpu.sync_copy(idx_hbm.at[pl.ds(base, TAIL)], idx_t)
            compute(idx_t, rows_t, TAIL)
            pltpu.sync_copy(rows_t, out_hbm.at[:, pl.ds(base, TAIL)])

    return gather_kernel


_gather = _make_sc_gather()


@jax.jit
def kernel(at_no, embed_ten):
    out_t = _gather(at_no.astype(jnp.int32), embed_ten.reshape(-1))
    return out_t.T
